# baseline (speedup 1.0000x reference)
import functools

import numpy as np
import jax
import jax.numpy as jnp
from jax.experimental import pallas as pl
from jax.experimental.pallas import tpu as pltpu

B = 1024
N = 20000
N_NUM = 32
N_FREQ = 48
D_EMB = 64
D_HIDDEN = 256
N_CLASSES = 10

K_GRP = 4
G = N_NUM // K_GRP
KIN = K_GRP * 2 * N_FREQ
KOUT = K_GRP * D_EMB
Z_COLS = N_NUM * N_FREQ

NB = 2000
N_BLOCKS = N // NB
QB = 512
Q_BLOCKS = B // QB

_EPS = 1e-7


def _enc_body(x_ref, sf_ref, wk_ref, bk_ref, w1g_ref, b1_ref, out_ref):
    r = x_ref.shape[0]
    z = jnp.dot(x_ref[...].astype(jnp.bfloat16), sf_ref[...],
                preferred_element_type=jnp.float32)
    acc = jnp.broadcast_to(b1_ref[...], (r, D_HIDDEN))
    for g in range(G):
        zg = z[:, g * K_GRP * N_FREQ:(g + 1) * K_GRP * N_FREQ]
        per = jnp.concatenate([jnp.cos(zg), jnp.sin(zg)], axis=1)
        h = jnp.dot(per.astype(jnp.bfloat16), wk_ref[g],
                    preferred_element_type=jnp.float32)
        h = jnp.maximum(h + bk_ref[g], 0.0)
        acc = acc + jnp.dot(h.astype(jnp.bfloat16), w1g_ref[g],
                            preferred_element_type=jnp.float32)
    out_ref[...] = jnp.maximum(acc, 0.0).astype(jnp.bfloat16)


def _encode(x, sf, wk, bk, w1g, b1, rows_per_block):
    rows = x.shape[0]
    grid = (rows // rows_per_block,)
    return pl.pallas_call(
        _enc_body,
        grid=grid,
        in_specs=[
            pl.BlockSpec((rows_per_block, N_NUM), lambda i: (i, 0)),
            pl.BlockSpec((N_NUM, Z_COLS), lambda i: (0, 0)),
            pl.BlockSpec((G, KIN, KOUT), lambda i: (0, 0, 0)),
            pl.BlockSpec((G, 1, KOUT), lambda i: (0, 0, 0)),
            pl.BlockSpec((G, KOUT, D_HIDDEN), lambda i: (0, 0, 0)),
            pl.BlockSpec((1, D_HIDDEN), lambda i: (0, 0)),
        ],
        out_specs=pl.BlockSpec((rows_per_block, D_HIDDEN), lambda i: (i, 0)),
        out_shape=jax.ShapeDtypeStruct((rows, D_HIDDEN), jnp.bfloat16),
        compiler_params=pltpu.CompilerParams(
            dimension_semantics=("parallel",)),
    )(x, sf, wk, bk, w1g, b1)


def _dist_body(qt_ref, c_ref, y_ref, out_ref, m_ref, acc_ref):
    nb = pl.program_id(1)
    c = c_ref[...]
    cf = c.astype(jnp.float32)
    c2 = jnp.sum(cf * cf, axis=1, keepdims=True)
    s = 2.0 * jnp.dot(c, qt_ref[...],
                      preferred_element_type=jnp.float32) - c2
    bm = jnp.max(s, axis=0, keepdims=True)
    m_prev = jnp.where(nb == 0, jnp.full_like(bm, -1e30), m_ref[0:1, :])
    m_new = jnp.maximum(m_prev, bm)
    e = jnp.exp(s - m_new)
    yrow = jnp.broadcast_to(y_ref[0], (16, NB))
    ridx = jax.lax.broadcasted_iota(jnp.int32, (16, NB), 0)
    ya = jnp.logical_or(ridx == yrow, ridx == N_CLASSES).astype(jnp.bfloat16)
    p = jnp.dot(ya, e.astype(jnp.bfloat16),
                preferred_element_type=jnp.float32)
    scale = jnp.exp(m_prev - m_new)
    acc_prev = jnp.where(nb == 0, jnp.zeros_like(acc_ref[...]), acc_ref[...])
    acc = acc_prev * scale + p
    m_ref[0:1, :] = m_new
    acc_ref[...] = acc

    @pl.when(nb == N_BLOCKS - 1)
    def _():
        denom = acc[N_CLASSES:N_CLASSES + 1, :]
        res = jnp.log(acc / denom + _EPS)
        ridx2 = jax.lax.broadcasted_iota(jnp.int32, res.shape, 0)
        out_ref[...] = jnp.where(ridx2 < N_CLASSES, res, 0.0)


def _distance(qt, cenc, y3):
    return pl.pallas_call(
        _dist_body,
        grid=(Q_BLOCKS, N_BLOCKS),
        in_specs=[
            pl.BlockSpec((D_HIDDEN, QB), lambda qb, nb: (0, qb)),
            pl.BlockSpec((NB, D_HIDDEN), lambda qb, nb: (nb, 0)),
            pl.BlockSpec((1, 1, NB), lambda qb, nb: (nb, 0, 0)),
        ],
        out_specs=pl.BlockSpec((16, QB), lambda qb, nb: (0, qb)),
        out_shape=jax.ShapeDtypeStruct((16, B), jnp.float32),
        scratch_shapes=[
            pltpu.VMEM((8, QB), jnp.float32),
            pltpu.VMEM((16, QB), jnp.float32),
        ],
        compiler_params=pltpu.CompilerParams(
            dimension_semantics=("parallel", "arbitrary")),
    )(qt, cenc, y3)


def kernel(x_num, candidate_x_num, candidate_y, freq, W_enc, b_enc, W1, b1):
    f32 = jnp.float32
    freq = freq.astype(f32)
    rows = jnp.arange(N_NUM)[:, None]
    cols = rows * N_FREQ + jnp.arange(N_FREQ)[None, :]
    sf = jnp.zeros((N_NUM, Z_COLS), f32).at[rows, cols].set(2.0 * np.pi * freq)
    sf = sf.astype(jnp.bfloat16)

    we = W_enc.astype(f32).reshape(G, K_GRP, 2, N_FREQ, D_EMB)
    wk = jnp.zeros((G, KIN, KOUT), f32)
    for j in range(K_GRP):
        rc = j * N_FREQ
        rs = K_GRP * N_FREQ + j * N_FREQ
        cc = j * D_EMB
        wk = wk.at[:, rc:rc + N_FREQ, cc:cc + D_EMB].set(we[:, j, 0])
        wk = wk.at[:, rs:rs + N_FREQ, cc:cc + D_EMB].set(we[:, j, 1])
    wk = wk.astype(jnp.bfloat16)

    bk = b_enc.astype(f32).reshape(G, 1, KOUT)
    w1g = W1.astype(f32).reshape(G, KOUT, D_HIDDEN).astype(jnp.bfloat16)
    b1r = b1.astype(f32).reshape(1, D_HIDDEN)

    qenc = _encode(x_num.astype(f32), sf, wk, bk, w1g, b1r, B // 2)
    cenc = _encode(candidate_x_num.astype(f32), sf, wk, bk, w1g, b1r, NB)

    qt = qenc.T
    y3 = candidate_y.astype(jnp.int32).reshape(N_BLOCKS, 1, NB)
    out = _distance(qt, cenc, y3)
    return out[:N_CLASSES, :].T
